# baseline (speedup 1.0000x reference)
import jax
import jax.numpy as jnp
from jax.experimental import pallas as pl
from jax.experimental.pallas import tpu as pltpu

_R, _C = 2048, 8192
_ROWS = _R - 8
_SIZES = [128] * 15 + [120]
_K = len(_SIZES)


def _dma_kernel(x_hbm, o_hbm, row_vmem, sems, sem_rin, sem_rout):
    copies = []
    start = 8
    for k, size in enumerate(_SIZES):
        cp = pltpu.make_async_copy(
            x_hbm.at[pl.ds(start, size), :], o_hbm.at[pl.ds(start, size), :],
            sems.at[k])
        cp.start()
        copies.append(cp)
        start += size
    rin = pltpu.make_async_copy(x_hbm.at[pl.ds(0, 8), :], row_vmem, sem_rin)
    rin.start()
    rin.wait()
    row = jax.lax.broadcasted_iota(jnp.int32, (8, _C), 0)
    col = jax.lax.broadcasted_iota(jnp.int32, (8, _C), 1)
    idx = row * _C + col
    x = row_vmem[...]
    row_vmem[...] = jnp.where(idx == 0, 10.0,
                    jnp.where(idx == 1, 30.0,
                    jnp.where(idx == 2, 20.0,
                    jnp.where(idx == 3, 40.0, x))))
    rout = pltpu.make_async_copy(row_vmem, o_hbm.at[pl.ds(0, 8), :], sem_rout)
    rout.start()
    for cp in copies:
        cp.wait()
    rout.wait()


def kernel(data):
    x = data.reshape(_R, _C)
    out = pl.pallas_call(
        _dma_kernel,
        in_specs=[pl.BlockSpec(memory_space=pl.ANY)],
        out_specs=pl.BlockSpec(memory_space=pl.ANY),
        out_shape=jax.ShapeDtypeStruct((_R, _C), jnp.float32),
        scratch_shapes=[pltpu.VMEM((8, _C), jnp.float32),
                        pltpu.SemaphoreType.DMA((_K,)),
                        pltpu.SemaphoreType.DMA,
                        pltpu.SemaphoreType.DMA],
    )(x)
    return out.reshape(-1)


# speedup vs baseline: 12.4222x; 12.4222x over previous
import jax
import jax.numpy as jnp
from jax.experimental import pallas as pl
from jax.experimental.pallas import tpu as pltpu

_R, _C = 2048, 8192
_BR = 256
_G = _R // _BR


def _copy_kernel(x_ref, o_ref):
    o_ref[...] = x_ref[...]

    @pl.when(pl.program_id(0) == 0)
    def _patch():
        row = jax.lax.broadcasted_iota(jnp.int32, (8, 128), 0)
        col = jax.lax.broadcasted_iota(jnp.int32, (8, 128), 1)
        x = x_ref[0:8, 0:128]
        patched = jnp.where(col == 0, 10.0,
                  jnp.where(col == 1, 30.0,
                  jnp.where(col == 2, 20.0,
                  jnp.where(col == 3, 40.0, x))))
        o_ref[0:8, 0:128] = jnp.where(row == 0, patched, x)


def kernel(data):
    x = data.reshape(_R, _C)
    out = pl.pallas_call(
        _copy_kernel,
        grid=(_G,),
        in_specs=[pl.BlockSpec((_BR, _C), lambda i: (i, 0))],
        out_specs=pl.BlockSpec((_BR, _C), lambda i: (i, 0)),
        out_shape=jax.ShapeDtypeStruct((_R, _C), jnp.float32),
    )(x)
    return out.reshape(-1)
